# Initial kernel scaffold; baseline (speedup 1.0000x reference)
import jax
import jax.numpy as jnp
from jax.experimental import pallas as pl


def kernel(query, neighbors, gumbel):
    raise NotImplementedError("write your pallas kernel here")


# speedup vs baseline: 7.0008x; 7.0008x over previous
import functools

import jax
import jax.numpy as jnp
import numpy as np
from jax.experimental import pallas as pl
from jax.experimental.pallas import tpu as pltpu

_K_SUBSET = 16
_EPS = float(np.finfo(np.float32).tiny)


def _dknn_block(q_ref, n_ref, g_ref, out_ref):
    q = q_ref[...]
    n = n_ref[...]
    g = g_ref[...]

    qsq = jnp.sum(q * q, axis=1, keepdims=True)
    nsq = jnp.sum(n * n, axis=1)[None, :]
    dot = jax.lax.dot_general(
        q, n, (((1,), (1,)), ((), ())),
        preferred_element_type=jnp.float32,
        precision=jax.lax.Precision.HIGHEST,
    )
    scores = (2.0 * dot - qsq - nsq) + g

    m0 = jnp.max(scores, axis=1, keepdims=True)
    h = jnp.exp((scores - m0) * 0.5)
    khot = jnp.zeros_like(h)
    for _ in range(_K_SUBSET):
        w = h * h
        denom = jnp.sum(w, axis=1, keepdims=True)
        onehot = w * (1.0 / denom)
        khot = khot + onehot
        h = h * jnp.sqrt(jnp.maximum(1.0 - onehot, _EPS))
        hmax = jnp.max(h, axis=1, keepdims=True)
        h = h * (1.0 / hmax)
    out_ref[...] = khot


@jax.jit
def kernel(query, neighbors, gumbel):
    Q, d = query.shape
    K = neighbors.shape[0]
    n_blocks = 2
    bq = Q // n_blocks
    return pl.pallas_call(
        _dknn_block,
        grid=(n_blocks,),
        in_specs=[
            pl.BlockSpec((bq, d), lambda i: (i, 0)),
            pl.BlockSpec((K, d), lambda i: (0, 0)),
            pl.BlockSpec((bq, K), lambda i: (i, 0)),
        ],
        out_specs=pl.BlockSpec((bq, K), lambda i: (i, 0)),
        out_shape=jax.ShapeDtypeStruct((Q, K), jnp.float32),
        compiler_params=pltpu.CompilerParams(
            dimension_semantics=("parallel",),
        ),
    )(query, neighbors, gumbel)
